# Initial kernel scaffold; baseline (speedup 1.0000x reference)
import jax
import jax.numpy as jnp
from jax.experimental import pallas as pl


def kernel(x_in, speakers, W_ih, W_hh, b_ih, b_hh):
    raise NotImplementedError("write your pallas kernel here")


# speedup vs baseline: 2.9072x; 2.9072x over previous
import functools

import jax
import jax.numpy as jnp
from jax.experimental import pallas as pl
from jax.experimental.pallas import tpu as pltpu

_B = 4096
_T = 50
_D_IN = 64
_D_MEM = 64
_NSPK = 10
_BLK = 256


def _body(x_ref, sp_ref, w_ref, b_ref, out_ref, mem_ref):
    mem_ref[...] = jnp.zeros_like(mem_ref)
    w = w_ref[...]
    b = b_ref[...]
    for t in range(_T):
        xt = x_ref[:, t, :]
        scol = sp_ref[:, t:t + 1]
        h = jnp.zeros((_BLK, _D_MEM), jnp.float32)
        masks = []
        for s in range(_NSPK):
            m = scol == s
            masks.append(m)
            h = jnp.where(m, mem_ref[s], h)
        hx = jnp.concatenate([xt, h], axis=1)
        g = jnp.dot(hx, w, preferred_element_type=jnp.float32) + b
        gi_r = g[:, 0:64]
        gi_z = g[:, 64:128]
        gi_n = g[:, 128:192]
        gh_r = g[:, 192:256]
        gh_z = g[:, 256:320]
        gh_n = g[:, 320:384]
        r = jax.nn.sigmoid(gi_r + gh_r)
        z = jax.nn.sigmoid(gi_z + gh_z)
        n = jnp.tanh(gi_n + r * gh_n)
        h_new = (1.0 - z) * n + z * h
        for s in range(_NSPK):
            mem_ref[s] = jnp.where(masks[s], h_new, mem_ref[s])
        out_ref[:, t, :] = h_new


@jax.jit
def kernel(x_in, speakers, W_ih, W_hh, b_ih, b_hh):
    d = _D_MEM
    w = jnp.zeros((_D_IN + d, 6 * d), jnp.float32)
    w = w.at[:_D_IN, :3 * d].set(W_ih.T)
    w = w.at[_D_IN:, 3 * d:].set(W_hh.T)
    b = jnp.concatenate([b_ih, b_hh])[None, :]
    sp = speakers.astype(jnp.int32)

    grid = (_B // _BLK,)
    out = pl.pallas_call(
        _body,
        grid=grid,
        in_specs=[
            pl.BlockSpec((_BLK, _T, _D_IN), lambda i: (i, 0, 0)),
            pl.BlockSpec((_BLK, _T), lambda i: (i, 0)),
            pl.BlockSpec((_D_IN + d, 6 * d), lambda i: (0, 0)),
            pl.BlockSpec((1, 6 * d), lambda i: (0, 0)),
        ],
        out_specs=pl.BlockSpec((_BLK, _T, d), lambda i: (i, 0, 0)),
        out_shape=jax.ShapeDtypeStruct((_B, _T, d), jnp.float32),
        scratch_shapes=[pltpu.VMEM((_NSPK, _BLK, d), jnp.float32)],
    )(x_in, sp, w, b)
    return out


# speedup vs baseline: 4.5485x; 1.5646x over previous
import functools

import jax
import jax.numpy as jnp
from jax.experimental import pallas as pl
from jax.experimental.pallas import tpu as pltpu

_B = 4096
_T = 50
_D_IN = 64
_D_MEM = 64
_NSPK = 10
_BLK = 512


def _body(x_ref, sp_ref, w_ref, b_ref, out_ref, mem_ref):
    mem_ref[...] = jnp.zeros_like(mem_ref)
    w = w_ref[...]
    b = b_ref[...]
    for t in range(_T):
        xt = x_ref[:, t * _D_IN:(t + 1) * _D_IN]
        scol = sp_ref[:, t:t + 1]
        h = jnp.zeros((_BLK, _D_MEM), jnp.float32)
        masks = []
        for s in range(_NSPK):
            m = scol == s
            masks.append(m)
            h = jnp.where(m, mem_ref[s], h)
        hx = jnp.concatenate([xt, h], axis=1)
        g = jnp.dot(hx, w, preferred_element_type=jnp.float32) + b
        gi_r = g[:, 0:64]
        gi_z = g[:, 64:128]
        gi_n = g[:, 128:192]
        gh_r = g[:, 192:256]
        gh_z = g[:, 256:320]
        gh_n = g[:, 320:384]
        r = jax.nn.sigmoid(gi_r + gh_r)
        z = jax.nn.sigmoid(gi_z + gh_z)
        n = jnp.tanh(gi_n + r * gh_n)
        h_new = (1.0 - z) * n + z * h
        for s in range(_NSPK):
            mem_ref[s] = jnp.where(masks[s], h_new, mem_ref[s])
        out_ref[:, t * _D_MEM:(t + 1) * _D_MEM] = h_new


@jax.jit
def kernel(x_in, speakers, W_ih, W_hh, b_ih, b_hh):
    d = _D_MEM
    w = jnp.zeros((_D_IN + d, 6 * d), jnp.float32)
    w = w.at[:_D_IN, :3 * d].set(W_ih.T)
    w = w.at[_D_IN:, 3 * d:].set(W_hh.T)
    b = jnp.concatenate([b_ih, b_hh])[None, :]
    sp = speakers.astype(jnp.int32)
    x2 = x_in.reshape(_B, _T * _D_IN)

    grid = (_B // _BLK,)
    out = pl.pallas_call(
        _body,
        grid=grid,
        in_specs=[
            pl.BlockSpec((_BLK, _T * _D_IN), lambda i: (i, 0)),
            pl.BlockSpec((_BLK, _T), lambda i: (i, 0)),
            pl.BlockSpec((_D_IN + d, 6 * d), lambda i: (0, 0)),
            pl.BlockSpec((1, 6 * d), lambda i: (0, 0)),
        ],
        out_specs=pl.BlockSpec((_BLK, _T * d), lambda i: (i, 0)),
        out_shape=jax.ShapeDtypeStruct((_B, _T * d), jnp.float32),
        compiler_params=pltpu.CompilerParams(
            dimension_semantics=("parallel",)),
        scratch_shapes=[pltpu.VMEM((_NSPK, _BLK, d), jnp.float32)],
    )(x2, sp, w, b)
    return out.reshape(_B, _T, d)


# speedup vs baseline: 28.8213x; 6.3365x over previous
import jax
import jax.numpy as jnp
from jax.experimental import pallas as pl
from jax.experimental.pallas import tpu as pltpu

_B = 4096
_T = 50
_D_IN = 64
_D_MEM = 64
_NSPK = 10
_BLK = 512


def _body(x_ref, sp_ref, w_ref, b_ref, out_ref, mem_ref):
    mem_ref[...] = jnp.zeros_like(mem_ref)
    w = w_ref[...]
    b = b_ref[...]
    d = _D_MEM
    for t in range(_T):
        xt = x_ref[t * _D_IN:(t + 1) * _D_IN, :]
        srow = sp_ref[t:t + 1, :]
        h = jnp.zeros((d, _BLK), jnp.float32)
        masks = []
        for s in range(_NSPK):
            m = srow == s
            masks.append(m)
            h = jnp.where(m, mem_ref[s], h)
        hx = jnp.concatenate([xt, h], axis=0)
        g = jax.lax.dot_general(
            w, hx, (((1,), (0,)), ((), ())),
            preferred_element_type=jnp.float32) + b
        r = jax.nn.sigmoid(g[0:d] + g[3 * d:4 * d])
        z = jax.nn.sigmoid(g[d:2 * d] + g[4 * d:5 * d])
        n = jnp.tanh(g[2 * d:3 * d] + r * g[5 * d:6 * d])
        h_new = (1.0 - z) * n + z * h
        for s in range(_NSPK):
            mem_ref[s] = jnp.where(masks[s], h_new, mem_ref[s])
        out_ref[t * d:(t + 1) * d, :] = h_new


@jax.jit
def kernel(x_in, speakers, W_ih, W_hh, b_ih, b_hh):
    d = _D_MEM
    w = jnp.zeros((6 * d, _D_IN + d), jnp.float32)
    w = w.at[:3 * d, :_D_IN].set(W_ih)
    w = w.at[3 * d:, _D_IN:].set(W_hh)
    b = jnp.concatenate([b_ih, b_hh])[:, None]
    xT = x_in.reshape(_B, _T * _D_IN).T
    spT = speakers.astype(jnp.int32).T

    grid = (_B // _BLK,)
    out = pl.pallas_call(
        _body,
        grid=grid,
        in_specs=[
            pl.BlockSpec((_T * _D_IN, _BLK), lambda i: (0, i)),
            pl.BlockSpec((_T, _BLK), lambda i: (0, i)),
            pl.BlockSpec((6 * d, _D_IN + d), lambda i: (0, 0)),
            pl.BlockSpec((6 * d, 1), lambda i: (0, 0)),
        ],
        out_specs=pl.BlockSpec((_T * d, _BLK), lambda i: (0, i)),
        out_shape=jax.ShapeDtypeStruct((_T * d, _B), jnp.float32),
        compiler_params=pltpu.CompilerParams(
            dimension_semantics=("parallel",)),
        scratch_shapes=[pltpu.VMEM((_NSPK, d, _BLK), jnp.float32)],
    )(xT, spT, w, b)
    return out.T.reshape(_B, _T, d)


# speedup vs baseline: 39.4052x; 1.3672x over previous
import jax
import jax.numpy as jnp
from jax.experimental import pallas as pl
from jax.experimental.pallas import tpu as pltpu

_B = 4096
_T = 50
_D_IN = 64
_D_MEM = 64
_NSPK = 10
_BLK = 1024


def _body(x_ref, sp_ref, w_ref, b_ref, out_ref, mem_ref):
    mem_ref[...] = jnp.zeros_like(mem_ref)
    w = w_ref[...]
    b = b_ref[...]
    d = _D_MEM
    for t in range(_T):
        xt = x_ref[t * _D_IN:(t + 1) * _D_IN, :]
        srow = sp_ref[t:t + 1, :]
        h = jnp.zeros((d, _BLK), jnp.float32)
        masks = []
        for s in range(_NSPK):
            m = srow == s
            masks.append(m)
            h = jnp.where(m, mem_ref[s], h)
        hx = jnp.concatenate([xt, h], axis=0)
        g = jax.lax.dot_general(
            w, hx, (((1,), (0,)), ((), ())),
            preferred_element_type=jnp.float32) + b
        r = jax.nn.sigmoid(g[0:d] + g[3 * d:4 * d])
        z = jax.nn.sigmoid(g[d:2 * d] + g[4 * d:5 * d])
        n = jnp.tanh(g[2 * d:3 * d] + r * g[5 * d:6 * d])
        h_new = (1.0 - z) * n + z * h
        for s in range(_NSPK):
            mem_ref[s] = jnp.where(masks[s], h_new, mem_ref[s])
        out_ref[t * d:(t + 1) * d, :] = h_new


@jax.jit
def kernel(x_in, speakers, W_ih, W_hh, b_ih, b_hh):
    d = _D_MEM
    w = jnp.zeros((6 * d, _D_IN + d), jnp.float32)
    w = w.at[:3 * d, :_D_IN].set(W_ih)
    w = w.at[3 * d:, _D_IN:].set(W_hh)
    b = jnp.concatenate([b_ih, b_hh])[:, None]
    xT = x_in.reshape(_B, _T * _D_IN).T
    spT = speakers.astype(jnp.int32).T

    grid = (_B // _BLK,)
    out = pl.pallas_call(
        _body,
        grid=grid,
        in_specs=[
            pl.BlockSpec((_T * _D_IN, _BLK), lambda i: (0, i)),
            pl.BlockSpec((_T, _BLK), lambda i: (0, i)),
            pl.BlockSpec((6 * d, _D_IN + d), lambda i: (0, 0)),
            pl.BlockSpec((6 * d, 1), lambda i: (0, 0)),
        ],
        out_specs=pl.BlockSpec((_T * d, _BLK), lambda i: (0, i)),
        out_shape=jax.ShapeDtypeStruct((_T * d, _B), jnp.float32),
        compiler_params=pltpu.CompilerParams(
            dimension_semantics=("parallel",),
            vmem_limit_bytes=115 * 1024 * 1024),
        scratch_shapes=[pltpu.VMEM((_NSPK, d, _BLK), jnp.float32)],
    )(xT, spT, w, b)
    return out.T.reshape(_B, _T, d)
